# baseline (speedup 1.0000x reference)
import functools

import jax
import jax.numpy as jnp
from jax import lax
from jax.experimental import pallas as pl
from jax.experimental.pallas import tpu as pltpu
from jax.experimental.pallas import tpu_sc as plsc

_B = 16384
_D = 64
_W = 128
_NCTX = 10
_NTGT = 6
_NC = 2
_NS = 16
_NW = _NC * _NS
_RPW = _B // _NW
_C = 32
_NCHUNK = _RPW // _C
_L = 16


def _row_compute(ctx_rows, tgt_rows, out_v, chunk_base, r):
    cb = r * _NCTX
    c = []
    for q in range(_D // _L):
        acc = ctx_rows[cb, pl.ds(q * _L, _L)]
        for j in range(1, _NCTX):
            acc = acc + ctx_rows[cb + j, pl.ds(q * _L, _L)]
        c.append(acc * (1.0 / _NCTX))
    tb = r * _NTGT
    lane = lax.iota(jnp.int32, _L)
    res = jnp.zeros((_L,), jnp.float32)
    for t in range(_NTGT):
        acc = c[0] * tgt_rows[tb + t, pl.ds(_D, _L)]
        for q in range(1, _D // _L):
            acc = acc + c[q] * tgt_rows[tb + t, pl.ds(_D + q * _L, _L)]
        res = jnp.where(lane == t, jnp.sum(acc), res)
    plsc.store_scatter(out_v, [lane + (chunk_base + r) * _NTGT], res,
                       mask=lane < _NTGT)


def _cbow_body(ctx_idx_hbm, tgt_idx_hbm, wide_hbm, out_hbm,
               ctx_idx_v, tgt_idx_v, ctx_rows, tgt_rows, out_v,
               sem_idx, sem_g):
    wid = lax.axis_index("s") * _NC + lax.axis_index("c")
    row0 = wid * _RPW

    cp0 = pltpu.async_copy(
        ctx_idx_hbm.at[pl.ds(row0 * _NCTX, _RPW * _NCTX)], ctx_idx_v, sem_idx)
    cp1 = pltpu.async_copy(
        tgt_idx_hbm.at[pl.ds(row0 * _NTGT, _RPW * _NTGT)], tgt_idx_v, sem_idx)
    cp0.wait()
    cp1.wait()

    for k in range(_NCHUNK):
        copies = []
        for s in range(_C * _NCTX // 64):
            idx = ctx_idx_v.at[pl.ds(k * _C * _NCTX + s * 64, 64)]
            copies.append(pltpu.async_copy(
                wide_hbm.at[idx], ctx_rows.at[pl.ds(s * 64, 64), :], sem_g))
        for s in range(_C * _NTGT // 64):
            idx = tgt_idx_v.at[pl.ds(k * _C * _NTGT + s * 64, 64)]
            copies.append(pltpu.async_copy(
                wide_hbm.at[idx], tgt_rows.at[pl.ds(s * 64, 64), :], sem_g))
        for cp in copies:
            cp.wait()

        def body(r, _):
            _row_compute(ctx_rows, tgt_rows, out_v, k * _C, r)
            return 0
        lax.fori_loop(0, _C, body, 0)

    pltpu.sync_copy(out_v, out_hbm.at[pl.ds(row0 * _NTGT, _RPW * _NTGT)])


_V = 1000001
_PBLK = 4096


def _pack_block(a_ref, b_ref, o_ref):
    o_ref[:, 0:_D] = a_ref[...]
    o_ref[:, _D:_W] = b_ref[...]


@jax.jit
def _pack(emb0, emb1):
    grid = (pl.cdiv(_V, _PBLK),)
    return pl.pallas_call(
        _pack_block,
        out_shape=jax.ShapeDtypeStruct((_V, _W), jnp.float32),
        grid=grid,
        in_specs=[
            pl.BlockSpec((_PBLK, _D), lambda i: (i, 0)),
            pl.BlockSpec((_PBLK, _D), lambda i: (i, 0)),
        ],
        out_specs=pl.BlockSpec((_PBLK, _W), lambda i: (i, 0)),
    )(emb0, emb1)


@jax.jit
def _cbow(ctx_idx, tgt_idx, wide):
    mesh = plsc.VectorSubcoreMesh(core_axis_name="c", subcore_axis_name="s")
    f = functools.partial(
        pl.kernel,
        out_type=jax.ShapeDtypeStruct((_B * _NTGT,), jnp.float32),
        mesh=mesh,
        compiler_params=pltpu.CompilerParams(
            needs_layout_passes=False, use_tc_tiling_on_sc=True),
        scratch_types=[
            pltpu.VMEM((_RPW * _NCTX,), jnp.int32),
            pltpu.VMEM((_RPW * _NTGT,), jnp.int32),
            pltpu.VMEM((_C * _NCTX, _W), jnp.float32),
            pltpu.VMEM((_C * _NTGT, _W), jnp.float32),
            pltpu.VMEM((_RPW * _NTGT,), jnp.float32),
            pltpu.SemaphoreType.DMA,
            pltpu.SemaphoreType.DMA,
        ],
    )(_cbow_body)
    return f(ctx_idx, tgt_idx, wide)


def kernel(data, emb0, emb1):
    data = data.astype(jnp.int32)
    ctx_idx = data[:, : _NCTX].reshape(-1)
    tgt_idx = data[:, _NCTX:].reshape(-1)
    wide = _pack(emb0, emb1)
    return _cbow(ctx_idx, tgt_idx, wide).reshape(_B, _NTGT)


# speedup vs baseline: 1.0164x; 1.0164x over previous
import functools

import jax
import jax.numpy as jnp
from jax import lax
from jax.experimental import pallas as pl
from jax.experimental.pallas import tpu as pltpu
from jax.experimental.pallas import tpu_sc as plsc

_B = 16384
_D = 64
_W = 128
_NCTX = 10
_NTGT = 6
_NC = 2
_NS = 16
_NW = _NC * _NS
_RPW = _B // _NW
_C = 32
_NCHUNK = _RPW // _C
_L = 16


def _row_compute(ctx_rows, tgt_rows, out_v, chunk_base, r):
    cb = r * _NCTX
    c = []
    for q in range(_D // _L):
        acc = ctx_rows[cb, pl.ds(q * _L, _L)]
        for j in range(1, _NCTX):
            acc = acc + ctx_rows[cb + j, pl.ds(q * _L, _L)]
        c.append(acc * (1.0 / _NCTX))
    tb = r * _NTGT
    lane = lax.iota(jnp.int32, _L)
    res = jnp.zeros((_L,), jnp.float32)
    for t in range(_NTGT):
        acc = c[0] * tgt_rows[tb + t, pl.ds(_D, _L)]
        for q in range(1, _D // _L):
            acc = acc + c[q] * tgt_rows[tb + t, pl.ds(_D + q * _L, _L)]
        res = jnp.where(lane == t, jnp.sum(acc), res)
    plsc.store_scatter(out_v, [lane + (chunk_base + r) * _NTGT], res,
                       mask=lane < _NTGT)


def _cbow_body(ctx_idx_hbm, tgt_idx_hbm, wide_hbm, out_hbm,
               ctx_idx_v, tgt_idx_v, ctx_rows, tgt_rows, out_v,
               sem_idx, sem_g):
    wid = lax.axis_index("s") * _NC + lax.axis_index("c")
    row0 = wid * _RPW

    cp0 = pltpu.async_copy(
        ctx_idx_hbm.at[pl.ds(row0 * _NCTX, _RPW * _NCTX)], ctx_idx_v, sem_idx)
    cp1 = pltpu.async_copy(
        tgt_idx_hbm.at[pl.ds(row0 * _NTGT, _RPW * _NTGT)], tgt_idx_v, sem_idx)
    cp0.wait()
    cp1.wait()

    for k in range(_NCHUNK):
        copies = []
        for s in range(_C * _NCTX // 64):
            idx = ctx_idx_v.at[pl.ds(k * _C * _NCTX + s * 64, 64)]
            copies.append(pltpu.async_copy(
                wide_hbm.at[idx], ctx_rows.at[pl.ds(s * 64, 64), :], sem_g))
        for s in range(_C * _NTGT // 64):
            idx = tgt_idx_v.at[pl.ds(k * _C * _NTGT + s * 64, 64)]
            copies.append(pltpu.async_copy(
                wide_hbm.at[idx], tgt_rows.at[pl.ds(s * 64, 64), :], sem_g))
        for cp in copies:
            cp.wait()

        def body(r, _):
            _row_compute(ctx_rows, tgt_rows, out_v, k * _C, r)
            return 0
        lax.fori_loop(0, _C, body, 0)

    pltpu.sync_copy(out_v, out_hbm.at[pl.ds(row0 * _NTGT, _RPW * _NTGT)])


_V = 1000001
_PBLK = 16384


def _pack_block(a_ref, b_ref, o_ref):
    o_ref[:, 0:_D] = a_ref[...]
    o_ref[:, _D:_W] = b_ref[...]


@jax.jit
def _pack(emb0, emb1):
    grid = (pl.cdiv(_V, _PBLK),)
    return pl.pallas_call(
        _pack_block,
        out_shape=jax.ShapeDtypeStruct((_V, _W), jnp.float32),
        grid=grid,
        in_specs=[
            pl.BlockSpec((_PBLK, _D), lambda i: (i, 0)),
            pl.BlockSpec((_PBLK, _D), lambda i: (i, 0)),
        ],
        out_specs=pl.BlockSpec((_PBLK, _W), lambda i: (i, 0)),
    )(emb0, emb1)


@jax.jit
def _cbow(ctx_idx, tgt_idx, wide):
    mesh = plsc.VectorSubcoreMesh(core_axis_name="c", subcore_axis_name="s")
    f = functools.partial(
        pl.kernel,
        out_type=jax.ShapeDtypeStruct((_B * _NTGT,), jnp.float32),
        mesh=mesh,
        compiler_params=pltpu.CompilerParams(
            needs_layout_passes=False, use_tc_tiling_on_sc=True),
        scratch_types=[
            pltpu.VMEM((_RPW * _NCTX,), jnp.int32),
            pltpu.VMEM((_RPW * _NTGT,), jnp.int32),
            pltpu.VMEM((_C * _NCTX, _W), jnp.float32),
            pltpu.VMEM((_C * _NTGT, _W), jnp.float32),
            pltpu.VMEM((_RPW * _NTGT,), jnp.float32),
            pltpu.SemaphoreType.DMA,
            pltpu.SemaphoreType.DMA,
        ],
    )(_cbow_body)
    return f(ctx_idx, tgt_idx, wide)


def kernel(data, emb0, emb1):
    data = data.astype(jnp.int32)
    ctx_idx = data[:, : _NCTX].reshape(-1)
    tgt_idx = data[:, _NCTX:].reshape(-1)
    wide = _pack(emb0, emb1)
    return _cbow(ctx_idx, tgt_idx, wide).reshape(_B, _NTGT)


# speedup vs baseline: 1.2840x; 1.2633x over previous
import functools

import jax
import jax.numpy as jnp
from jax import lax
from jax.experimental import pallas as pl
from jax.experimental.pallas import tpu as pltpu
from jax.experimental.pallas import tpu_sc as plsc

_B = 16384
_D = 64
_W = 128
_NCTX = 10
_NTGT = 6
_NC = 2
_NS = 16
_NW = _NC * _NS
_RPW = _B // _NW
_C = 32
_NCHUNK = _RPW // _C
_L = 16


def _row_compute(ctx_rows, tgt_rows, out_v, chunk_base, r):
    cb = r * _NCTX
    c = []
    for q in range(_D // _L):
        acc = ctx_rows[cb, pl.ds(q * _L, _L)]
        for j in range(1, _NCTX):
            acc = acc + ctx_rows[cb + j, pl.ds(q * _L, _L)]
        c.append(acc * (1.0 / _NCTX))
    tb = r * _NTGT
    lane = lax.iota(jnp.int32, _L)
    res = jnp.zeros((_L,), jnp.float32)
    for t in range(_NTGT):
        acc = c[0] * tgt_rows[tb + t, pl.ds(_D, _L)]
        for q in range(1, _D // _L):
            acc = acc + c[q] * tgt_rows[tb + t, pl.ds(_D + q * _L, _L)]
        res = jnp.where(lane == t, jnp.sum(acc), res)
    plsc.store_scatter(out_v, [lane + (chunk_base + r) * _NTGT], res,
                       mask=lane < _NTGT)


def _cbow_body(ctx_idx_hbm, tgt_idx_hbm, wide_hbm, out_hbm,
               ctx_idx_v, tgt_idx_v, ctx_rows, tgt_rows, out_v,
               sem_idx, sem_g):
    wid = lax.axis_index("s") * _NC + lax.axis_index("c")
    row0 = wid * _RPW

    cp0 = pltpu.async_copy(
        ctx_idx_hbm.at[pl.ds(row0 * _NCTX, _RPW * _NCTX)], ctx_idx_v, sem_idx)
    cp1 = pltpu.async_copy(
        tgt_idx_hbm.at[pl.ds(row0 * _NTGT, _RPW * _NTGT)], tgt_idx_v, sem_idx)
    cp0.wait()
    cp1.wait()

    for k in range(_NCHUNK):
        copies = []
        for s in range(_C * _NCTX // 64):
            idx = ctx_idx_v.at[pl.ds(k * _C * _NCTX + s * 64, 64)]
            copies.append(pltpu.async_copy(
                wide_hbm.at[idx], ctx_rows.at[pl.ds(s * 64, 64), :], sem_g))
        for s in range(_C * _NTGT // 64):
            idx = tgt_idx_v.at[pl.ds(k * _C * _NTGT + s * 64, 64)]
            copies.append(pltpu.async_copy(
                wide_hbm.at[idx], tgt_rows.at[pl.ds(s * 64, 64), :], sem_g))
        for cp in copies:
            cp.wait()

        def body(r, _):
            _row_compute(ctx_rows, tgt_rows, out_v, k * _C, r)
            return 0
        lax.fori_loop(0, _C, body, 0)

    pltpu.sync_copy(out_v, out_hbm.at[pl.ds(row0 * _NTGT, _RPW * _NTGT)])


_V = 1000001
_PBLK = 16384


def _pack_block(a_ref, b_ref, o_ref):
    o_ref[:, 0:_D] = a_ref[...]
    o_ref[:, _D:_W] = b_ref[...]


@jax.jit
def _pack(emb0, emb1):
    grid = (pl.cdiv(_V, _PBLK),)
    return pl.pallas_call(
        _pack_block,
        out_shape=jax.ShapeDtypeStruct((_V, _W), jnp.float32),
        grid=grid,
        in_specs=[
            pl.BlockSpec((_PBLK, _D), lambda i: (i, 0)),
            pl.BlockSpec((_PBLK, _D), lambda i: (i, 0)),
        ],
        out_specs=pl.BlockSpec((_PBLK, _W), lambda i: (i, 0)),
    )(emb0, emb1)


@jax.jit
def _cbow(ctx_idx, tgt_idx, wide):
    mesh = plsc.VectorSubcoreMesh(core_axis_name="c", subcore_axis_name="s")
    f = functools.partial(
        pl.kernel,
        out_type=jax.ShapeDtypeStruct((_B * _NTGT,), jnp.float32),
        mesh=mesh,
        compiler_params=pltpu.CompilerParams(
            needs_layout_passes=False, use_tc_tiling_on_sc=True),
        scratch_types=[
            pltpu.VMEM((_RPW * _NCTX,), jnp.int32),
            pltpu.VMEM((_RPW * _NTGT,), jnp.int32),
            pltpu.VMEM((_C * _NCTX, _W), jnp.float32),
            pltpu.VMEM((_C * _NTGT, _W), jnp.float32),
            pltpu.VMEM((_RPW * _NTGT,), jnp.float32),
            pltpu.SemaphoreType.DMA,
            pltpu.SemaphoreType.DMA,
        ],
    )(_cbow_body)
    return f(ctx_idx, tgt_idx, wide)


def kernel(data, emb0, emb1):
    data = data.astype(jnp.int32)
    ctx_idx = data[:, : _NCTX].reshape(-1)
    tgt_idx = data[:, _NCTX:].reshape(-1)
    wide = jnp.concatenate([emb0, emb1], axis=1)
    return _cbow(ctx_idx, tgt_idx, wide).reshape(_B, _NTGT)


# speedup vs baseline: 1.3177x; 1.0263x over previous
import functools

import jax
import jax.numpy as jnp
from jax import lax
from jax.experimental import pallas as pl
from jax.experimental.pallas import tpu as pltpu
from jax.experimental.pallas import tpu_sc as plsc

_B = 16384
_D = 64
_W = 128
_NCTX = 10
_NTGT = 6
_NC = 2
_NS = 16
_NW = _NC * _NS
_RPW = _B // _NW
_C = 16
_NCHUNK = _RPW // _C
_L = 16
_CSTR = 80


def _row_compute(ctx_rows, tgt_rows, out_v, chunk_base, r):
    cb = r * _NCTX
    c = []
    for q in range(_D // _L):
        acc = ctx_rows[cb, pl.ds(q * _L, _L)]
        for j in range(1, _NCTX):
            acc = acc + ctx_rows[cb + j, pl.ds(q * _L, _L)]
        c.append(acc * (1.0 / _NCTX))
    tb = r * _NTGT
    lane = lax.iota(jnp.int32, _L)
    res = jnp.zeros((_L,), jnp.float32)
    for t in range(_NTGT):
        acc = c[0] * tgt_rows[tb + t, pl.ds(_D, _L)]
        for q in range(1, _D // _L):
            acc = acc + c[q] * tgt_rows[tb + t, pl.ds(_D + q * _L, _L)]
        res = jnp.where(lane == t, jnp.sum(acc), res)
    plsc.store_scatter(out_v, [lane + (chunk_base + r) * _NTGT], res,
                       mask=lane < _NTGT)


def _cbow_body(ctx_idx_hbm, tgt_idx_hbm, wide_hbm, out_hbm,
               ctx_idx_v, tgt_idx_v, ctx_rows, tgt_rows, out_v,
               sem_idx, sem_g):
    wid = lax.axis_index("s") * _NC + lax.axis_index("c")
    row0 = wid * _RPW

    cp0 = pltpu.async_copy(
        ctx_idx_hbm.at[pl.ds(row0 * _NCTX, _RPW * _NCTX)], ctx_idx_v, sem_idx)
    cp1 = pltpu.async_copy(
        tgt_idx_hbm.at[pl.ds(row0 * _NTGT, _RPW * _NTGT)], tgt_idx_v, sem_idx)
    cp0.wait()
    cp1.wait()

    def fire(k, b):
        cps = []
        for s in range(_C * _NCTX // _CSTR):
            idx = ctx_idx_v.at[pl.ds(k * _C * _NCTX + s * _CSTR, _CSTR)]
            cps.append(pltpu.async_copy(
                wide_hbm.at[idx], ctx_rows.at[b, pl.ds(s * _CSTR, _CSTR), :],
                sem_g))
        idx = tgt_idx_v.at[pl.ds(k * _C * _NTGT, _C * _NTGT)]
        cps.append(pltpu.async_copy(
            wide_hbm.at[idx], tgt_rows.at[b], sem_g))
        return cps

    pend = [None, None]
    pend[0] = fire(0, 0)
    for k in range(_NCHUNK):
        b = k & 1
        if k + 1 < _NCHUNK:
            pend[1 - b] = fire(k + 1, 1 - b)
        for cp in pend[b]:
            cp.wait()

        crb = ctx_rows.at[b]
        trb = tgt_rows.at[b]

        def body(r, _):
            _row_compute(crb, trb, out_v, k * _C, r)
            return 0
        lax.fori_loop(0, _C, body, 0)

    pltpu.sync_copy(out_v, out_hbm.at[pl.ds(row0 * _NTGT, _RPW * _NTGT)])


_V = 1000001
_PBLK = 16384


def _pack_block(a_ref, b_ref, o_ref):
    o_ref[:, 0:_D] = a_ref[...]
    o_ref[:, _D:_W] = b_ref[...]


@jax.jit
def _pack(emb0, emb1):
    grid = (pl.cdiv(_V, _PBLK),)
    return pl.pallas_call(
        _pack_block,
        out_shape=jax.ShapeDtypeStruct((_V, _W), jnp.float32),
        grid=grid,
        in_specs=[
            pl.BlockSpec((_PBLK, _D), lambda i: (i, 0)),
            pl.BlockSpec((_PBLK, _D), lambda i: (i, 0)),
        ],
        out_specs=pl.BlockSpec((_PBLK, _W), lambda i: (i, 0)),
    )(emb0, emb1)


@jax.jit
def _cbow(ctx_idx, tgt_idx, wide):
    mesh = plsc.VectorSubcoreMesh(core_axis_name="c", subcore_axis_name="s")
    f = functools.partial(
        pl.kernel,
        out_type=jax.ShapeDtypeStruct((_B * _NTGT,), jnp.float32),
        mesh=mesh,
        compiler_params=pltpu.CompilerParams(
            needs_layout_passes=False, use_tc_tiling_on_sc=True),
        scratch_types=[
            pltpu.VMEM((_RPW * _NCTX,), jnp.int32),
            pltpu.VMEM((_RPW * _NTGT,), jnp.int32),
            pltpu.VMEM((2, _C * _NCTX, _W), jnp.float32),
            pltpu.VMEM((2, _C * _NTGT, _W), jnp.float32),
            pltpu.VMEM((_RPW * _NTGT,), jnp.float32),
            pltpu.SemaphoreType.DMA,
            pltpu.SemaphoreType.DMA,
        ],
    )(_cbow_body)
    return f(ctx_idx, tgt_idx, wide)


def kernel(data, emb0, emb1):
    data = data.astype(jnp.int32)
    ctx_idx = data[:, : _NCTX].reshape(-1)
    tgt_idx = data[:, _NCTX:].reshape(-1)
    wide = jnp.concatenate([emb0, emb1], axis=1)
    return _cbow(ctx_idx, tgt_idx, wide).reshape(_B, _NTGT)
